# baseline (speedup 1.0000x reference)
import functools

import jax
import jax.numpy as jnp
from jax import lax
from jax.experimental import pallas as pl
from jax.experimental.pallas import tpu as pltpu
from jax.experimental.pallas import tpu_sc as plsc

_BATCH = 4096
_HIST = 50
_D = 64
_B = _BATCH * _HIST
_NC = 2
_NS = 16
_NW = _NC * _NS
_BPW = _B // _NW
_CH = 128
_NCH = _BPW // _CH

_mesh = plsc.VectorSubcoreMesh(core_axis_name="c", subcore_axis_name="s")


@functools.partial(
    pl.kernel,
    mesh=_mesh,
    out_type=jax.ShapeDtypeStruct((_B, _D), jnp.float32),
    compiler_params=pltpu.CompilerParams(use_tc_tiling_on_sc=False),
    scratch_types=[
        pltpu.VMEM((_NCH, _CH), jnp.int32),
        pltpu.VMEM((_CH, _D), jnp.float32),
        pltpu.SemaphoreType.DMA,
    ],
)
def _gather(idx_hbm, table_hbm, out_hbm, idx_v, rows_v, sem):
    wid = lax.axis_index("s") * _NC + lax.axis_index("c")
    pltpu.sync_copy(idx_hbm.at[wid], idx_v)

    def body(j, carry):
        pltpu.async_copy(table_hbm.at[idx_v.at[j]], rows_v, sem).wait()
        pltpu.sync_copy(rows_v, out_hbm.at[pl.ds(wid * _BPW + j * _CH, _CH)])
        return carry

    lax.fori_loop(0, _NCH, body, 0)


def kernel(word_batch, table):
    idx = word_batch.astype(jnp.int32).reshape(_NW, _NCH, _CH)
    out = _gather(idx, table)
    return out.reshape(_BATCH, _HIST, _D)


# speedup vs baseline: 1.0445x; 1.0445x over previous
import functools

import jax
import jax.numpy as jnp
from jax import lax
from jax.experimental import pallas as pl
from jax.experimental.pallas import tpu as pltpu
from jax.experimental.pallas import tpu_sc as plsc

_BATCH = 4096
_HIST = 50
_D = 64
_B = _BATCH * _HIST
_NC = 2
_NS = 16
_NW = _NC * _NS
_BPW = _B // _NW
_CH = 128
_NCH = _BPW // _CH
_NB = 10

_mesh = plsc.VectorSubcoreMesh(core_axis_name="c", subcore_axis_name="s")


@functools.partial(
    pl.kernel,
    mesh=_mesh,
    out_type=jax.ShapeDtypeStruct((_B, _D), jnp.float32),
    compiler_params=pltpu.CompilerParams(use_tc_tiling_on_sc=False),
    scratch_types=[
        pltpu.VMEM((_NCH, _CH), jnp.int32),
        pltpu.VMEM((_NB * _CH, _D), jnp.float32),
    ] + [pltpu.SemaphoreType.DMA] * _NB,
)
def _gather(idx_hbm, table_hbm, out_hbm, idx_v, rows_v, *sems):
    wid = lax.axis_index("s") * _NC + lax.axis_index("c")
    pltpu.sync_copy(idx_hbm.at[wid], idx_v)

    def buf(b):
        return rows_v.at[pl.ds(b * _CH, _CH)]

    for b in range(_NB):
        pltpu.async_copy(table_hbm.at[idx_v.at[b]], buf(b), sems[b])

    def grp(g, carry):
        for b in range(_NB):
            chunk = g * _NB + b
            pltpu.make_async_copy(table_hbm.at[idx_v.at[b]], buf(b), sems[b]).wait()
            pltpu.sync_copy(buf(b), out_hbm.at[pl.ds(wid * _BPW + chunk * _CH, _CH)])
            pltpu.async_copy(table_hbm.at[idx_v.at[chunk + _NB]], buf(b), sems[b])
        return carry

    lax.fori_loop(0, _NCH // _NB - 1, grp, 0)

    for b in range(_NB):
        chunk = _NCH - _NB + b
        pltpu.make_async_copy(table_hbm.at[idx_v.at[b]], buf(b), sems[b]).wait()
        pltpu.sync_copy(buf(b), out_hbm.at[pl.ds(wid * _BPW + chunk * _CH, _CH)])


def kernel(word_batch, table):
    idx = word_batch.astype(jnp.int32).reshape(_NW, _NCH, _CH)
    out = _gather(idx, table)
    return out.reshape(_BATCH, _HIST, _D)
